# baseline (speedup 1.0000x reference)
import jax
import jax.numpy as jnp
from jax.experimental import pallas as pl
from jax.experimental.pallas import tpu as pltpu


def _pad1(x):
    return jnp.pad(x, ((1, 1), (1, 1), (0, 0)))


def _epi(acc, scale, shift, cout, residual, relu):
    y = acc * scale[0, :cout] + shift[0, :cout]
    if residual is not None:
        y = y + residual
    if relu:
        y = jnp.maximum(y, 0.0)
    return y


def _conv3x3_s1(xp, w, scale, shift, cin, cout, ho, wo,
                residual=None, relu=True):
    acc = None
    for di in range(3):
        rows = xp[di:di + ho]
        cat = jnp.concatenate([rows[:, dj:dj + wo] for dj in range(3)],
                              axis=-1).reshape(ho * wo, 3 * cin)
        part = jnp.dot(cat, w[di * 3 * cin:(di + 1) * 3 * cin, :cout],
                       preferred_element_type=jnp.float32)
        acc = part if acc is None else acc + part
    return _epi(acc, scale, shift, cout, residual, relu)


def _conv3x3_s2_phases(ph, w, scale, shift, cin, cout, ho, wo):
    def phase(pr, pc):
        c0 = (2 * pr + pc) * cin
        return ph[..., c0:c0 + cin]

    acc = None
    for di in range(3):
        cat = jnp.concatenate(
            [phase(di % 2, dj % 2)[di // 2:di // 2 + ho,
                                   dj // 2:dj // 2 + wo] for dj in range(3)],
            axis=-1).reshape(ho * wo, 3 * cin)
        part = jnp.dot(cat, w[di * 3 * cin:(di + 1) * 3 * cin, :cout],
                       preferred_element_type=jnp.float32)
        acc = part if acc is None else acc + part
    return _epi(acc, scale, shift, cout, None, True)


def _block_s1(x_bf, identity_f32, p1, p2, c, ho, wo):
    y1 = _conv3x3_s1(_pad1(x_bf), *p1, c, c, ho, wo)
    y1_bf = y1.astype(jnp.bfloat16).reshape(ho, wo, c)
    y2 = _conv3x3_s1(_pad1(y1_bf), *p2, c, c, ho, wo, residual=identity_f32)
    return y2.astype(jnp.bfloat16).reshape(ho, wo, c), y2


def _block_down(ph, p1, p2, pdown, cin, cout, ho, wo):
    dw, ds, dt = pdown
    a = ph[..., 3 * cin:4 * cin][0:ho, 0:wo].reshape(ho * wo, cin)
    ident = jnp.dot(a, dw[:cin, :cout], preferred_element_type=jnp.float32)
    ident = ident * pdown[1][0, :cout] + pdown[2][0, :cout]
    y1 = _conv3x3_s2_phases(ph, *p1, cin, cout, ho, wo)
    y1_bf = y1.astype(jnp.bfloat16).reshape(ho, wo, cout)
    y2 = _conv3x3_s1(_pad1(y1_bf), *p2, cout, cout, ho, wo, residual=ident)
    return y2.astype(jnp.bfloat16).reshape(ho, wo, cout), y2


def _deint_w(x, c):
    w2 = x.shape[1] // 2
    t = jnp.swapaxes(x, 0, 1).reshape(w2, 2, x.shape[0], c)
    return jnp.swapaxes(t[:, 0], 0, 1), jnp.swapaxes(t[:, 1], 0, 1)


def _phase_split_k(x, c):
    h2 = x.shape[0] // 2
    r = x.reshape(h2, 2, x.shape[1], c)
    ee, eo = _deint_w(r[:, 0], c)
    oe, oo = _deint_w(r[:, 1], c)
    return jnp.concatenate([ee, eo, oe, oo], axis=-1)


def _pa_body(ph_ref, stem_w, stem_s, stem_t,
             l1a1w, l1a1s, l1a1t, l1a2w, l1a2s, l1a2t,
             l1b1w, l1b1s, l1b1t, l1b2w, l1b2s, l1b2t,
             a1w, a1s, a1t, a2w, a2s, a2t, dw, ds, dt,
             b1w, b1s, b1t, b2w, b2s, b2t, out_ref):
    ph = ph_ref[0]
    cat = jnp.concatenate(
        [ph[di:di + 112, dj:dj + 112] for di in range(4) for dj in range(4)],
        axis=-1).reshape(12544, 192)
    y = jnp.dot(cat, stem_w[:, :64], preferred_element_type=jnp.float32)
    y = jnp.maximum(y * stem_s[0, :64] + stem_t[0, :64], 0.0)
    mp = _pad1(y.reshape(112, 112, 64))
    pool = None
    for di in range(3):
        for dj in range(3):
            s = mp[di:di + 112, dj:dj + 112]
            pool = s if pool is None else jnp.maximum(pool, s)
    pool = pool.reshape(56, 2, 112, 64)[:, 0]
    pool = _deint_w(pool, 64)[0]

    ident = pool.reshape(3136, 64)
    x_bf = pool.astype(jnp.bfloat16)
    x_bf, x_f32 = _block_s1(x_bf, ident, (l1a1w, l1a1s, l1a1t),
                            (l1a2w, l1a2s, l1a2t), 64, 56, 56)
    x_bf, _ = _block_s1(x_bf, x_f32, (l1b1w, l1b1s, l1b1t),
                        (l1b2w, l1b2s, l1b2t), 64, 56, 56)

    ph2 = _phase_split_k(_pad1(x_bf), 64)
    x_bf, x_f32 = _block_down(ph2, (a1w, a1s, a1t), (a2w, a2s, a2t),
                              (dw, ds, dt), 64, 128, 28, 28)
    x_bf, _ = _block_s1(x_bf, x_f32, (b1w, b1s, b1t), (b2w, b2s, b2t),
                        128, 28, 28)
    out_ref[0] = x_bf


def _pb_body(x_ref,
             a1w, a1s, a1t, a2w, a2s, a2t, dw, ds, dt,
             b1w, b1s, b1t, b2w, b2s, b2t,
             c1w, c1s, c1t, c2w, c2s, c2t, ew, es, et,
             d1w, d1s, d1t, d2w, d2s, d2t, feat_ref):
    feats = []
    for k in range(4):
        ph3 = _phase_split_k(_pad1(x_ref[k]), 128)
        x_bf, x_f32 = _block_down(ph3, (a1w, a1s, a1t), (a2w, a2s, a2t),
                                  (dw, ds, dt), 128, 256, 14, 14)
        x_bf, _ = _block_s1(x_bf, x_f32, (b1w, b1s, b1t), (b2w, b2s, b2t),
                            256, 14, 14)
        ph4 = _phase_split_k(_pad1(x_bf), 256)
        x_bf, x_f32 = _block_down(ph4, (c1w, c1s, c1t), (c2w, c2s, c2t),
                                  (ew, es, et), 256, 512, 7, 7)
        _, x_f32 = _block_s1(x_bf, x_f32, (d1w, d1s, d1t), (d2w, d2s, d2t),
                             512, 7, 7)
        feats.append(jnp.mean(x_f32, axis=0, keepdims=True))
    feat_ref[...] = jnp.concatenate(feats, axis=0).reshape(4, 1, 512)


def _full_spec(shape):
    n = len(shape)
    return pl.BlockSpec(shape, lambda i, _n=n: (0,) * _n)


def _call(body, inputs, params, in_block, out_shape, out_block, grid):
    nb = len(in_block)
    return pl.pallas_call(
        body,
        out_shape=jax.ShapeDtypeStruct(out_shape[0], out_shape[1]),
        grid=(grid,),
        in_specs=[pl.BlockSpec(in_block, lambda i, _n=nb: (i,) + (0,) * (_n - 1))]
                 + [_full_spec(p.shape) for p in params],
        out_specs=pl.BlockSpec(out_block,
                               lambda i, _n=len(out_block): (i,) + (0,) * (_n - 1)),
        compiler_params=pltpu.CompilerParams(
            dimension_semantics=("parallel",)),
    )(inputs, *params)


def _stem_phases(x):
    xp = jnp.pad(x.astype(jnp.bfloat16), ((0, 0), (0, 0), (3, 3), (3, 3)))
    ph = jnp.concatenate([xp[:, :, pr::2, pc::2]
                          for pr in range(2) for pc in range(2)], axis=1)
    return jnp.transpose(ph, (0, 2, 3, 1))


def _stem_weight(stem_w):
    idx = []
    for Di in range(4):
        for Dj in range(4):
            for pr in range(2):
                for pc in range(2):
                    for ci in range(3):
                        di, dj = 2 * Di + pr, 2 * Dj + pc
                        idx.append((di * 7 + dj) * 3 + ci
                                   if di < 7 and dj < 7 else 200)
    return jnp.take(stem_w, jnp.array(idx, jnp.int32), axis=0)


def _phase_split(x):
    xp = jnp.pad(x, ((0, 0), (1, 1), (1, 1), (0, 0)))
    return jnp.concatenate([xp[:, pr::2, pc::2, :]
                            for pr in range(2) for pc in range(2)], axis=-1)


def kernel(x, stem_w, stem_scale, stem_shift, layer1_b0_conv1_w, layer1_b0_conv1_scale, layer1_b0_conv1_shift, layer1_b0_conv2_w, layer1_b0_conv2_scale, layer1_b0_conv2_shift, layer1_b1_conv1_w, layer1_b1_conv1_scale, layer1_b1_conv1_shift, layer1_b1_conv2_w, layer1_b1_conv2_scale, layer1_b1_conv2_shift, layer2_b0_conv1_w, layer2_b0_conv1_scale, layer2_b0_conv1_shift, layer2_b0_conv2_w, layer2_b0_conv2_scale, layer2_b0_conv2_shift, layer2_b0_down_w, layer2_b0_down_scale, layer2_b0_down_shift, layer2_b1_conv1_w, layer2_b1_conv1_scale, layer2_b1_conv1_shift, layer2_b1_conv2_w, layer2_b1_conv2_scale, layer2_b1_conv2_shift, layer3_b0_conv1_w, layer3_b0_conv1_scale, layer3_b0_conv1_shift, layer3_b0_conv2_w, layer3_b0_conv2_scale, layer3_b0_conv2_shift, layer3_b0_down_w, layer3_b0_down_scale, layer3_b0_down_shift, layer3_b1_conv1_w, layer3_b1_conv1_scale, layer3_b1_conv1_shift, layer3_b1_conv2_w, layer3_b1_conv2_scale, layer3_b1_conv2_shift, layer4_b0_conv1_w, layer4_b0_conv1_scale, layer4_b0_conv1_shift, layer4_b0_conv2_w, layer4_b0_conv2_scale, layer4_b0_conv2_shift, layer4_b0_down_w, layer4_b0_down_scale, layer4_b0_down_shift, layer4_b1_conv1_w, layer4_b1_conv1_scale, layer4_b1_conv1_shift, layer4_b1_conv2_w, layer4_b1_conv2_scale, layer4_b1_conv2_shift, fc_w, fc_b):
    stem_ph = _stem_phases(x)

    pa_params = [_stem_weight(stem_w), stem_scale, stem_shift,
                 layer1_b0_conv1_w, layer1_b0_conv1_scale, layer1_b0_conv1_shift,
                 layer1_b0_conv2_w, layer1_b0_conv2_scale, layer1_b0_conv2_shift,
                 layer1_b1_conv1_w, layer1_b1_conv1_scale, layer1_b1_conv1_shift,
                 layer1_b1_conv2_w, layer1_b1_conv2_scale, layer1_b1_conv2_shift,
                 layer2_b0_conv1_w, layer2_b0_conv1_scale, layer2_b0_conv1_shift,
                 layer2_b0_conv2_w, layer2_b0_conv2_scale, layer2_b0_conv2_shift,
                 layer2_b0_down_w, layer2_b0_down_scale, layer2_b0_down_shift,
                 layer2_b1_conv1_w, layer2_b1_conv1_scale, layer2_b1_conv1_shift,
                 layer2_b1_conv2_w, layer2_b1_conv2_scale, layer2_b1_conv2_shift]
    l2_out = _call(_pa_body, stem_ph, pa_params,
                   (1, 115, 115, 12), ((32, 28, 28, 128), jnp.bfloat16),
                   (1, 28, 28, 128), 32)

    pb_params = [layer3_b0_conv1_w, layer3_b0_conv1_scale, layer3_b0_conv1_shift,
                 layer3_b0_conv2_w, layer3_b0_conv2_scale, layer3_b0_conv2_shift,
                 layer3_b0_down_w, layer3_b0_down_scale, layer3_b0_down_shift,
                 layer3_b1_conv1_w, layer3_b1_conv1_scale, layer3_b1_conv1_shift,
                 layer3_b1_conv2_w, layer3_b1_conv2_scale, layer3_b1_conv2_shift,
                 layer4_b0_conv1_w, layer4_b0_conv1_scale, layer4_b0_conv1_shift,
                 layer4_b0_conv2_w, layer4_b0_conv2_scale, layer4_b0_conv2_shift,
                 layer4_b0_down_w, layer4_b0_down_scale, layer4_b0_down_shift,
                 layer4_b1_conv1_w, layer4_b1_conv1_scale, layer4_b1_conv1_shift,
                 layer4_b1_conv2_w, layer4_b1_conv2_scale, layer4_b1_conv2_shift]
    feat = _call(_pb_body, l2_out, pb_params,
                 (4, 28, 28, 128), ((32, 1, 512), jnp.float32),
                 (4, 1, 512), 8)

    return feat.reshape(32, 512) @ fc_w.T + fc_b[None, :]


# speedup vs baseline: 1.8865x; 1.8865x over previous
import jax
import jax.numpy as jnp
from jax.experimental import pallas as pl
from jax.experimental.pallas import tpu as pltpu


def _pad1(x):
    return jnp.pad(x, ((1, 1), (1, 1), (0, 0)))


def _epi(acc, scale, shift, cout, residual, relu):
    y = acc * scale[0, :cout] + shift[0, :cout]
    if residual is not None:
        y = y + residual
    if relu:
        y = jnp.maximum(y, 0.0)
    return y


def _conv3x3_s1(xp, w, scale, shift, cin, cout, ho, wo,
                residual=None, relu=True):
    acc = None
    for di in range(3):
        rows = xp[di:di + ho]
        cat = jnp.concatenate([rows[:, dj:dj + wo] for dj in range(3)],
                              axis=-1).reshape(ho * wo, 3 * cin)
        part = jnp.dot(cat, w[di * 3 * cin:(di + 1) * 3 * cin, :cout],
                       preferred_element_type=jnp.float32)
        acc = part if acc is None else acc + part
    return _epi(acc, scale, shift, cout, residual, relu)


def _conv3x3_s2_phases(ph, w, scale, shift, cin, cout, ho, wo):
    def phase(pr, pc):
        c0 = (2 * pr + pc) * cin
        return ph[..., c0:c0 + cin]

    acc = None
    for di in range(3):
        cat = jnp.concatenate(
            [phase(di % 2, dj % 2)[di // 2:di // 2 + ho,
                                   dj // 2:dj // 2 + wo] for dj in range(3)],
            axis=-1).reshape(ho * wo, 3 * cin)
        part = jnp.dot(cat, w[di * 3 * cin:(di + 1) * 3 * cin, :cout],
                       preferred_element_type=jnp.float32)
        acc = part if acc is None else acc + part
    return _epi(acc, scale, shift, cout, None, True)


def _block_s1(x_bf, identity_f32, p1, p2, c, ho, wo):
    y1 = _conv3x3_s1(_pad1(x_bf), *p1, c, c, ho, wo)
    y1_bf = y1.astype(jnp.bfloat16).reshape(ho, wo, c)
    y2 = _conv3x3_s1(_pad1(y1_bf), *p2, c, c, ho, wo, residual=identity_f32)
    return y2.astype(jnp.bfloat16).reshape(ho, wo, c), y2


def _block_down(ph, p1, p2, pdown, cin, cout, ho, wo):
    dw, ds, dt = pdown
    a = ph[..., 3 * cin:4 * cin][0:ho, 0:wo].reshape(ho * wo, cin)
    ident = jnp.dot(a, dw[:cin, :cout], preferred_element_type=jnp.float32)
    ident = ident * pdown[1][0, :cout] + pdown[2][0, :cout]
    y1 = _conv3x3_s2_phases(ph, *p1, cin, cout, ho, wo)
    y1_bf = y1.astype(jnp.bfloat16).reshape(ho, wo, cout)
    y2 = _conv3x3_s1(_pad1(y1_bf), *p2, cout, cout, ho, wo, residual=ident)
    return y2.astype(jnp.bfloat16).reshape(ho, wo, cout), y2


def _deint_w(x, c):
    w2 = x.shape[1] // 2
    t = jnp.swapaxes(x, 0, 1).reshape(w2, 2, x.shape[0], c)
    return jnp.swapaxes(t[:, 0], 0, 1), jnp.swapaxes(t[:, 1], 0, 1)


def _phase_split_k(x, c):
    h2 = x.shape[0] // 2
    r = x.reshape(h2, 2, x.shape[1], c)
    ee, eo = _deint_w(r[:, 0], c)
    oe, oo = _deint_w(r[:, 1], c)
    return jnp.concatenate([ee, eo, oe, oo], axis=-1)


def _pa_body(x_ref, stem_w, stem_s, stem_t,
             l1a1w, l1a1s, l1a1t, l1a2w, l1a2s, l1a2t,
             l1b1w, l1b1s, l1b1t, l1b2w, l1b2s, l1b2t,
             a1w, a1s, a1t, a2w, a2s, a2t, dw, ds, dt,
             b1w, b1s, b1t, b2w, b2s, b2t, out_ref):
    x3 = x_ref[0].astype(jnp.bfloat16)
    xp = jnp.pad(x3, ((0, 0), (3, 3), (3, 3)))
    a = jnp.swapaxes(xp, 0, 1).reshape(115, 2, 3, 230)
    planes = []
    for pr in range(2):
        for pc in range(2):
            b = jnp.swapaxes(a[:, pr], 1, 2)
            b = jnp.swapaxes(b, 0, 1).reshape(115, 2, 115, 3)[:, pc]
            planes.append(jnp.swapaxes(b, 0, 1))
    ph = jnp.concatenate(planes, axis=-1)

    cat = jnp.concatenate(
        [ph[di:di + 112, dj:dj + 112] for di in range(4) for dj in range(4)],
        axis=-1).reshape(12544, 192)
    y = jnp.dot(cat, stem_w[:, :64], preferred_element_type=jnp.float32)
    y = jnp.maximum(y * stem_s[0, :64] + stem_t[0, :64], 0.0)
    mp = _pad1(y.reshape(112, 112, 64))
    pool = None
    for di in range(3):
        for dj in range(3):
            s = mp[di:di + 112, dj:dj + 112]
            pool = s if pool is None else jnp.maximum(pool, s)
    pool = pool.reshape(56, 2, 112, 64)[:, 0]
    pool = _deint_w(pool, 64)[0]

    ident = pool.reshape(3136, 64)
    x_bf = pool.astype(jnp.bfloat16)
    x_bf, x_f32 = _block_s1(x_bf, ident, (l1a1w, l1a1s, l1a1t),
                            (l1a2w, l1a2s, l1a2t), 64, 56, 56)
    x_bf, _ = _block_s1(x_bf, x_f32, (l1b1w, l1b1s, l1b1t),
                        (l1b2w, l1b2s, l1b2t), 64, 56, 56)

    ph2 = _phase_split_k(_pad1(x_bf), 64)
    x_bf, x_f32 = _block_down(ph2, (a1w, a1s, a1t), (a2w, a2s, a2t),
                              (dw, ds, dt), 64, 128, 28, 28)
    x_bf, _ = _block_s1(x_bf, x_f32, (b1w, b1s, b1t), (b2w, b2s, b2t),
                        128, 28, 28)
    out_ref[0] = x_bf


def _pb_body(x_ref,
             a1w, a1s, a1t, a2w, a2s, a2t, dw, ds, dt,
             b1w, b1s, b1t, b2w, b2s, b2t,
             c1w, c1s, c1t, c2w, c2s, c2t, ew, es, et,
             d1w, d1s, d1t, d2w, d2s, d2t, feat_ref):
    feats = []
    for k in range(4):
        ph3 = _phase_split_k(_pad1(x_ref[k]), 128)
        x_bf, x_f32 = _block_down(ph3, (a1w, a1s, a1t), (a2w, a2s, a2t),
                                  (dw, ds, dt), 128, 256, 14, 14)
        x_bf, _ = _block_s1(x_bf, x_f32, (b1w, b1s, b1t), (b2w, b2s, b2t),
                            256, 14, 14)
        ph4 = _phase_split_k(_pad1(x_bf), 256)
        x_bf, x_f32 = _block_down(ph4, (c1w, c1s, c1t), (c2w, c2s, c2t),
                                  (ew, es, et), 256, 512, 7, 7)
        _, x_f32 = _block_s1(x_bf, x_f32, (d1w, d1s, d1t), (d2w, d2s, d2t),
                             512, 7, 7)
        feats.append(jnp.mean(x_f32, axis=0, keepdims=True))
    feat_ref[...] = jnp.concatenate(feats, axis=0).reshape(4, 1, 512)


def _full_spec(shape):
    n = len(shape)
    return pl.BlockSpec(shape, lambda i, _n=n: (0,) * _n)


def _call(body, inputs, params, in_block, out_shape, out_block, grid):
    nb = len(in_block)
    return pl.pallas_call(
        body,
        out_shape=jax.ShapeDtypeStruct(out_shape[0], out_shape[1]),
        grid=(grid,),
        in_specs=[pl.BlockSpec(in_block, lambda i, _n=nb: (i,) + (0,) * (_n - 1))]
                 + [_full_spec(p.shape) for p in params],
        out_specs=pl.BlockSpec(out_block,
                               lambda i, _n=len(out_block): (i,) + (0,) * (_n - 1)),
        compiler_params=pltpu.CompilerParams(
            dimension_semantics=("parallel",)),
    )(inputs, *params)


def _stem_weight(stem_w):
    idx = []
    for Di in range(4):
        for Dj in range(4):
            for pr in range(2):
                for pc in range(2):
                    for ci in range(3):
                        di, dj = 2 * Di + pr, 2 * Dj + pc
                        idx.append((di * 7 + dj) * 3 + ci
                                   if di < 7 and dj < 7 else 200)
    return jnp.take(stem_w, jnp.array(idx, jnp.int32), axis=0)


def _phase_split(x):
    xp = jnp.pad(x, ((0, 0), (1, 1), (1, 1), (0, 0)))
    return jnp.concatenate([xp[:, pr::2, pc::2, :]
                            for pr in range(2) for pc in range(2)], axis=-1)


def kernel(x, stem_w, stem_scale, stem_shift, layer1_b0_conv1_w, layer1_b0_conv1_scale, layer1_b0_conv1_shift, layer1_b0_conv2_w, layer1_b0_conv2_scale, layer1_b0_conv2_shift, layer1_b1_conv1_w, layer1_b1_conv1_scale, layer1_b1_conv1_shift, layer1_b1_conv2_w, layer1_b1_conv2_scale, layer1_b1_conv2_shift, layer2_b0_conv1_w, layer2_b0_conv1_scale, layer2_b0_conv1_shift, layer2_b0_conv2_w, layer2_b0_conv2_scale, layer2_b0_conv2_shift, layer2_b0_down_w, layer2_b0_down_scale, layer2_b0_down_shift, layer2_b1_conv1_w, layer2_b1_conv1_scale, layer2_b1_conv1_shift, layer2_b1_conv2_w, layer2_b1_conv2_scale, layer2_b1_conv2_shift, layer3_b0_conv1_w, layer3_b0_conv1_scale, layer3_b0_conv1_shift, layer3_b0_conv2_w, layer3_b0_conv2_scale, layer3_b0_conv2_shift, layer3_b0_down_w, layer3_b0_down_scale, layer3_b0_down_shift, layer3_b1_conv1_w, layer3_b1_conv1_scale, layer3_b1_conv1_shift, layer3_b1_conv2_w, layer3_b1_conv2_scale, layer3_b1_conv2_shift, layer4_b0_conv1_w, layer4_b0_conv1_scale, layer4_b0_conv1_shift, layer4_b0_conv2_w, layer4_b0_conv2_scale, layer4_b0_conv2_shift, layer4_b0_down_w, layer4_b0_down_scale, layer4_b0_down_shift, layer4_b1_conv1_w, layer4_b1_conv1_scale, layer4_b1_conv1_shift, layer4_b1_conv2_w, layer4_b1_conv2_scale, layer4_b1_conv2_shift, fc_w, fc_b):
    pa_params = [_stem_weight(stem_w), stem_scale, stem_shift,
                 layer1_b0_conv1_w, layer1_b0_conv1_scale, layer1_b0_conv1_shift,
                 layer1_b0_conv2_w, layer1_b0_conv2_scale, layer1_b0_conv2_shift,
                 layer1_b1_conv1_w, layer1_b1_conv1_scale, layer1_b1_conv1_shift,
                 layer1_b1_conv2_w, layer1_b1_conv2_scale, layer1_b1_conv2_shift,
                 layer2_b0_conv1_w, layer2_b0_conv1_scale, layer2_b0_conv1_shift,
                 layer2_b0_conv2_w, layer2_b0_conv2_scale, layer2_b0_conv2_shift,
                 layer2_b0_down_w, layer2_b0_down_scale, layer2_b0_down_shift,
                 layer2_b1_conv1_w, layer2_b1_conv1_scale, layer2_b1_conv1_shift,
                 layer2_b1_conv2_w, layer2_b1_conv2_scale, layer2_b1_conv2_shift]
    l2_out = _call(_pa_body, x, pa_params,
                   (1, 3, 224, 224), ((32, 28, 28, 128), jnp.bfloat16),
                   (1, 28, 28, 128), 32)

    pb_params = [layer3_b0_conv1_w, layer3_b0_conv1_scale, layer3_b0_conv1_shift,
                 layer3_b0_conv2_w, layer3_b0_conv2_scale, layer3_b0_conv2_shift,
                 layer3_b0_down_w, layer3_b0_down_scale, layer3_b0_down_shift,
                 layer3_b1_conv1_w, layer3_b1_conv1_scale, layer3_b1_conv1_shift,
                 layer3_b1_conv2_w, layer3_b1_conv2_scale, layer3_b1_conv2_shift,
                 layer4_b0_conv1_w, layer4_b0_conv1_scale, layer4_b0_conv1_shift,
                 layer4_b0_conv2_w, layer4_b0_conv2_scale, layer4_b0_conv2_shift,
                 layer4_b0_down_w, layer4_b0_down_scale, layer4_b0_down_shift,
                 layer4_b1_conv1_w, layer4_b1_conv1_scale, layer4_b1_conv1_shift,
                 layer4_b1_conv2_w, layer4_b1_conv2_scale, layer4_b1_conv2_shift]
    feat = _call(_pb_body, l2_out, pb_params,
                 (4, 28, 28, 128), ((32, 1, 512), jnp.float32),
                 (4, 1, 512), 8)

    return feat.reshape(32, 512) @ fc_w.T + fc_b[None, :]
